# baseline (speedup 1.0000x reference)
import functools

import jax
import jax.numpy as jnp
from jax import lax
from jax.experimental import pallas as pl
from jax.experimental.pallas import tpu as pltpu
from jax.experimental.pallas import tpu_sc as plsc

_N_NODES = 50000
_N_EDGES = 1600000
_N_UNARY = 8
_N_BINARY = 2

_NC = 2
_NS = 16
_NW = _NC * _NS

_B_PER_W = _N_EDGES // _NW
_U_MAIN = 1568
_U_LAST = _N_NODES - 31 * _U_MAIN


def _sc_copy(u_hbm, b_hbm, uo_hbm, bo_hbm, sem_u, sem_b):
    wid = lax.axis_index("s") * _NC + lax.axis_index("c")
    b_base = wid * _B_PER_W
    u_base = wid * _U_MAIN

    cb = pltpu.make_async_copy(
        b_hbm.at[pl.ds(b_base, _B_PER_W)],
        bo_hbm.at[pl.ds(b_base, _B_PER_W)],
        sem_b,
    )
    cb.start()

    def copy_unary(rows):
        cu = pltpu.make_async_copy(
            u_hbm.at[pl.ds(u_base, rows)],
            uo_hbm.at[pl.ds(u_base, rows)],
            sem_u,
        )
        cu.start()
        cu.wait()

    @pl.when(wid < _NW - 1)
    def _():
        copy_unary(_U_MAIN)

    @pl.when(wid == _NW - 1)
    def _():
        copy_unary(_U_LAST)

    cb.wait()


def kernel(unary, binary, index1, index2):
    mesh = plsc.VectorSubcoreMesh(core_axis_name="c", subcore_axis_name="s")
    run = functools.partial(
        pl.kernel,
        mesh=mesh,
        out_type=[
            jax.ShapeDtypeStruct(unary.shape, unary.dtype),
            jax.ShapeDtypeStruct(binary.shape, binary.dtype),
        ],
        scratch_types=[
            pltpu.SemaphoreType.DMA,
            pltpu.SemaphoreType.DMA,
        ],
        compiler_params=pltpu.CompilerParams(use_tc_tiling_on_sc=False),
    )(_sc_copy)
    return tuple(run(unary, binary))


# speedup vs baseline: 1.4542x; 1.4542x over previous
import functools

import jax
import jax.numpy as jnp
from jax import lax
from jax.experimental import pallas as pl
from jax.experimental.pallas import tpu as pltpu
from jax.experimental.pallas import tpu_sc as plsc

_N_NODES = 50000
_N_EDGES = 1600000
_N_UNARY = 8
_N_BINARY = 2

_NC = 2
_NS = 16
_NW = _NC * _NS

_B_PER_W = _N_EDGES // _NW
_B_CHUNK = 6250
_B_NCHUNK = _B_PER_W // _B_CHUNK

_U_MAIN = 1568
_U_LAST = _N_NODES - 31 * _U_MAIN


def _sc_copy(u_hbm, b_hbm, uo_hbm, bo_hbm, u_buf, b_buf0, b_buf1, sem_u, sem_b):
    wid = lax.axis_index("s") * _NC + lax.axis_index("c")
    b_base = wid * _B_PER_W
    u_base = wid * _U_MAIN
    bufs = (b_buf0, b_buf1)

    def start_in(k):
        pltpu.make_async_copy(
            b_hbm.at[pl.ds(b_base + k * _B_CHUNK, _B_CHUNK)],
            bufs[k % 2],
            sem_b,
        ).start()

    def wait_in(k):
        pltpu.make_async_copy(
            b_hbm.at[pl.ds(b_base + k * _B_CHUNK, _B_CHUNK)],
            bufs[k % 2],
            sem_b,
        ).wait()

    start_in(0)
    start_in(1)

    def copy_unary(rows):
        cu = pltpu.make_async_copy(
            u_hbm.at[pl.ds(u_base, rows)], u_buf.at[pl.ds(0, rows)], sem_u
        )
        cu.start()
        cu.wait()
        pltpu.sync_copy(
            u_buf.at[pl.ds(0, rows)], uo_hbm.at[pl.ds(u_base, rows)]
        )

    @pl.when(wid < _NW - 1)
    def _():
        copy_unary(_U_MAIN)

    @pl.when(wid == _NW - 1)
    def _():
        copy_unary(_U_LAST)

    for k in range(_B_NCHUNK):
        wait_in(k)
        pltpu.sync_copy(
            bufs[k % 2],
            bo_hbm.at[pl.ds(b_base + k * _B_CHUNK, _B_CHUNK)],
        )
        if k + 2 < _B_NCHUNK:
            start_in(k + 2)


def kernel(unary, binary, index1, index2):
    mesh = plsc.VectorSubcoreMesh(core_axis_name="c", subcore_axis_name="s")
    run = functools.partial(
        pl.kernel,
        mesh=mesh,
        out_type=[
            jax.ShapeDtypeStruct(unary.shape, unary.dtype),
            jax.ShapeDtypeStruct(binary.shape, binary.dtype),
        ],
        scratch_types=[
            pltpu.VMEM((_U_MAIN, _N_UNARY), jnp.float32),
            pltpu.VMEM((_B_CHUNK, _N_BINARY), jnp.float32),
            pltpu.VMEM((_B_CHUNK, _N_BINARY), jnp.float32),
            pltpu.SemaphoreType.DMA,
            pltpu.SemaphoreType.DMA,
        ],
        compiler_params=pltpu.CompilerParams(use_tc_tiling_on_sc=False),
    )(_sc_copy)
    return tuple(run(unary, binary))


# speedup vs baseline: 3.5637x; 2.4506x over previous
import functools

import jax
import jax.numpy as jnp
from jax import lax
from jax.experimental import pallas as pl
from jax.experimental.pallas import tpu as pltpu
from jax.experimental.pallas import tpu_sc as plsc

_N_NODES = 50000
_N_EDGES = 1600000
_N_UNARY = 8
_N_BINARY = 2

_NC = 2
_NS = 16
_NW = _NC * _NS

_B_PER_W = _N_EDGES // _NW
_B_CHUNK = 400
_B_NCHUNK = _B_PER_W // _B_CHUNK

_U_WORKERS = 25
_U_MAIN = _N_NODES // _U_WORKERS
_U_CHUNK = 200
_U_NCHUNK = _U_MAIN // _U_CHUNK


def _sc_copy(u_hbm, b_hbm, uo_hbm, bo_hbm, u_buf, b_buf0, b_buf1, sem_u, sem_b):
    wid = lax.axis_index("s") * _NC + lax.axis_index("c")
    b_base = wid * _B_PER_W
    u_base = wid * _U_MAIN
    bufs = (b_buf0, b_buf1)

    def b_copy(k, buf):
        return pltpu.make_async_copy(
            b_hbm.at[pl.ds(b_base + k * _B_CHUNK, _B_CHUNK)], buf, sem_b
        )

    b_copy(0, bufs[0]).start()
    b_copy(1, bufs[1]).start()

    def copy_unary(n_chunks):
        for j in range(n_chunks):
            cu = pltpu.make_async_copy(
                u_hbm.at[pl.ds(u_base + j * _U_CHUNK, _U_CHUNK)], u_buf, sem_u
            )
            cu.start()
            cu.wait()
            pltpu.sync_copy(
                u_buf, uo_hbm.at[pl.ds(u_base + j * _U_CHUNK, _U_CHUNK)]
            )

    @pl.when(wid < _U_WORKERS)
    def _():
        copy_unary(_U_NCHUNK)

    for k in range(_B_NCHUNK):
        b_copy(k, bufs[k % 2]).wait()
        pltpu.sync_copy(
            bufs[k % 2], bo_hbm.at[pl.ds(b_base + k * _B_CHUNK, _B_CHUNK)]
        )
        if k + 2 < _B_NCHUNK:
            b_copy(k + 2, bufs[k % 2]).start()


def kernel(unary, binary, index1, index2):
    mesh = plsc.VectorSubcoreMesh(core_axis_name="c", subcore_axis_name="s")
    run = functools.partial(
        pl.kernel,
        mesh=mesh,
        out_type=[
            jax.ShapeDtypeStruct(unary.shape, unary.dtype),
            jax.ShapeDtypeStruct(binary.shape, binary.dtype),
        ],
        scratch_types=[
            pltpu.VMEM((_U_CHUNK, _N_UNARY), jnp.float32),
            pltpu.VMEM((_B_CHUNK, _N_BINARY), jnp.float32),
            pltpu.VMEM((_B_CHUNK, _N_BINARY), jnp.float32),
            pltpu.SemaphoreType.DMA,
            pltpu.SemaphoreType.DMA,
        ],
        compiler_params=pltpu.CompilerParams(use_tc_tiling_on_sc=True),
    )(_sc_copy)
    return tuple(run(unary, binary))


# speedup vs baseline: 3.8330x; 1.0756x over previous
import jax
import jax.numpy as jnp
from jax.experimental import pallas as pl
from jax.experimental.pallas import tpu as pltpu

_N_NODES = 50000
_N_EDGES = 1600000
_N_UNARY = 8
_N_BINARY = 2

_GRID = 125
_B_BLOCK = _N_EDGES // _GRID
_U_BLOCK = _N_NODES // _GRID


def _copy_body(u_ref, b_ref, uo_ref, bo_ref):
    uo_ref[...] = u_ref[...]
    bo_ref[...] = b_ref[...]


def kernel(unary, binary, index1, index2):
    uo, bo = pl.pallas_call(
        _copy_body,
        grid=(_GRID,),
        in_specs=[
            pl.BlockSpec((_U_BLOCK, _N_UNARY), lambda i: (i, 0)),
            pl.BlockSpec((_B_BLOCK, _N_BINARY), lambda i: (i, 0)),
        ],
        out_specs=[
            pl.BlockSpec((_U_BLOCK, _N_UNARY), lambda i: (i, 0)),
            pl.BlockSpec((_B_BLOCK, _N_BINARY), lambda i: (i, 0)),
        ],
        out_shape=[
            jax.ShapeDtypeStruct(unary.shape, unary.dtype),
            jax.ShapeDtypeStruct(binary.shape, binary.dtype),
        ],
        compiler_params=pltpu.CompilerParams(
            dimension_semantics=("arbitrary",),
        ),
    )(unary, binary)
    return (uo, bo)
